# baseline (speedup 1.0000x reference)
import functools

import jax
import jax.numpy as jnp
from jax import lax
from jax.experimental import pallas as pl
from jax.experimental.pallas import tpu as pltpu
from jax.experimental.pallas import tpu_sc as plsc

N = 10000
D = 128
E = 320000

NC = 2
NS = 16
NW = NC * NS

QH = 80
NP = QH * 128
CH = 128
NCH = 80
BI = 16
NBLK = NCH // BI
EPW = NCH * CH
EP = EPW * NW
RPT = NP // NS

_f32 = jnp.float32
_i32 = jnp.int32


def _h_body(dst_hbm, deg_hbm, hist_v, idxd_v, ident_v, hist_sh):
    c = lax.axis_index("c")
    s = lax.axis_index("s")
    w = c * NS + s
    zero16 = jnp.zeros((16,), _i32)
    one16 = jnp.ones((16,), _i32)

    @pl.loop(0, 5)
    def _(k):
        ident_v[0, pl.ds(k * 16, 16)] = (
            lax.broadcasted_iota(_i32, (16,), 0) + k * 16)

    for i in range(3):
        @pl.loop(0, QH)
        def _(q):
            @pl.loop(0, 8)
            def _(l):
                hist_v[q, pl.ds(l * 16, 16)] = zero16

        @pl.when(s == 0)
        def _():
            pltpu.sync_copy(hist_v, hist_sh)

        plsc.subcore_barrier()

        pltpu.sync_copy(dst_hbm.at[i, w], idxd_v)

        @pl.loop(0, NCH)
        def _(q):
            @pl.loop(0, 8)
            def _(l):
                v = idxd_v[q, pl.ds(l * 16, 16)]
                plsc.addupdate_scatter(hist_v, [v >> 7, v & 127], one16)

        pltpu.sync_copy(hist_v, hist_sh.at[ident_v.at[0]], add=True)
        plsc.subcore_barrier()

        @pl.when(s == 0)
        def _():
            pltpu.sync_copy(hist_sh, deg_hbm.at[i, c])

        plsc.subcore_barrier()


_h_call = functools.partial(
    pl.kernel,
    out_type=jax.ShapeDtypeStruct((3, NC, QH, 128), _i32),
    compiler_params=pltpu.CompilerParams(needs_layout_passes=False),
    mesh=plsc.VectorSubcoreMesh(core_axis_name="c", subcore_axis_name="s"),
    scratch_types=[
        pltpu.VMEM((QH, 128), _i32),
        pltpu.VMEM((NCH, 128), _i32),
        pltpu.VMEM((1, QH), _i32),
        pltpu.VMEM_SHARED((QH, 128), _i32),
    ],
)(_h_body)


def _a_body(x0b, x1b, x2b, w1, w2, w3, degb, y0b, y1b, y2b, disb):
    d = (degb[:, 0, :] + degb[:, 1, :] + 1).astype(_f32)
    dis = lax.rsqrt(d)
    disb[...] = dis
    for xb, wb, yb, i in ((x0b, w1, y0b, 0), (x1b, w2, y1b, 1),
                          (x2b, w3, y2b, 2)):
        xw = jnp.dot(xb[...], wb[...], preferred_element_type=_f32)
        yb[...] = xw * dis[i][:, None]


def _a_call(xp, ws, deg_flat):
    RA = NP // 8
    grid = 8
    return pl.pallas_call(
        _a_body,
        grid=(grid,),
        in_specs=[
            pl.BlockSpec((RA, D), lambda r: (r, 0)),
            pl.BlockSpec((RA, D), lambda r: (r, 0)),
            pl.BlockSpec((RA, D), lambda r: (r, 0)),
            pl.BlockSpec((D, D), lambda r: (0, 0)),
            pl.BlockSpec((D, D), lambda r: (0, 0)),
            pl.BlockSpec((D, D), lambda r: (0, 0)),
            pl.BlockSpec((3, NC, RA), lambda r: (0, 0, r)),
        ],
        out_specs=[
            pl.BlockSpec((RA, D), lambda r: (r, 0)),
            pl.BlockSpec((RA, D), lambda r: (r, 0)),
            pl.BlockSpec((RA, D), lambda r: (r, 0)),
            pl.BlockSpec((3, RA), lambda r: (0, r)),
        ],
        out_shape=[
            jax.ShapeDtypeStruct((NP, D), _f32),
            jax.ShapeDtypeStruct((NP, D), _f32),
            jax.ShapeDtypeStruct((NP, D), _f32),
            jax.ShapeDtypeStruct((3, NP), _f32),
        ],
    )(xp[0], xp[1], xp[2], ws[0], ws[1], ws[2], deg_flat)


def _b_body(y0, y1, y2, src_hbm, dst_hbm, out_hbm,
            idxs_v, idxd_v, buf_v, bufb_v, z_sh, sem, semb, semi):
    c = lax.axis_index("c")
    s = lax.axis_index("s")
    w = c * NS + s
    zero16 = jnp.zeros((16,), _f32)

    def zero_my_rows():
        @pl.loop(0, CH)
        def _(q):
            @pl.loop(0, 8)
            def _(l):
                buf_v[q, pl.ds(l * 16, 16)] = zero16

        @pl.loop(0, RPT // CH)
        def _(q):
            pltpu.sync_copy(buf_v, z_sh.at[pl.ds(s * RPT + q * CH, CH)])

    zero_my_rows()
    plsc.subcore_barrier()

    for i, y_ref in enumerate((y0, y1, y2)):
        def stage(b, pb, sync):
            blk = pl.ds(b * BI, BI)
            if sync:
                pltpu.sync_copy(src_hbm.at[i, w, blk], idxs_v.at[pb])
                pltpu.sync_copy(dst_hbm.at[i, w, blk], idxd_v.at[pb])
            else:
                pltpu.async_copy(src_hbm.at[i, w, blk], idxs_v.at[pb], semi)
                pltpu.async_copy(dst_hbm.at[i, w, blk], idxd_v.at[pb], semi)

        def wait_stage(pb):
            pltpu.make_async_copy(
                src_hbm.at[i, w, pl.ds(0, BI)], idxs_v.at[pb], semi).wait()
            pltpu.make_async_copy(
                dst_hbm.at[i, w, pl.ds(0, BI)], idxd_v.at[pb], semi).wait()

        def fire(pb, j, buf, sm):
            pltpu.async_copy(y_ref.at[idxs_v.at[pb, j]], buf, sm)

        def wait_g(buf, sm):
            pltpu.make_async_copy(
                y_ref.at[idxs_v.at[0, 0]], buf, sm).wait()

        def scat(pb, j, buf):
            pltpu.sync_copy(buf, z_sh.at[idxd_v.at[pb, j]], add=True)

        stage(0, 0, True)
        for b in range(NBLK):
            pb = b % 2
            if b + 1 < NBLK:
                stage(b + 1, 1 - pb, False)
            fire(pb, 0, buf_v, sem)
            for t in range(BI // 2):
                j = 2 * t
                fire(pb, j + 1, bufb_v, semb)
                wait_g(buf_v, sem)
                scat(pb, j, buf_v)
                if j + 2 < BI:
                    fire(pb, j + 2, buf_v, sem)
                wait_g(bufb_v, semb)
                scat(pb, j + 1, bufb_v)
            if b + 1 < NBLK:
                wait_stage(1 - pb)

        plsc.subcore_barrier()

        for cc in range(NC):
            @pl.when(c == cc)
            def _():
                pltpu.sync_copy(z_sh.at[pl.ds(s * RPT, RPT)],
                                out_hbm.at[i, cc, pl.ds(s * RPT, RPT)])

        if i < 2:
            zero_my_rows()

        plsc.subcore_barrier()


_b_call = functools.partial(
    pl.kernel,
    out_type=jax.ShapeDtypeStruct((3, NC, NP, D), _f32),
    compiler_params=pltpu.CompilerParams(needs_layout_passes=False),
    mesh=plsc.VectorSubcoreMesh(core_axis_name="c", subcore_axis_name="s"),
    scratch_types=[
        pltpu.VMEM((2, BI, 128), _i32),
        pltpu.VMEM((2, BI, 128), _i32),
        pltpu.VMEM((CH, D), _f32),
        pltpu.VMEM((CH, D), _f32),
        pltpu.VMEM_SHARED((NP, D), _f32),
        pltpu.SemaphoreType.DMA,
        pltpu.SemaphoreType.DMA,
        pltpu.SemaphoreType.DMA,
    ],
)(_b_body)


_RC = NP // 16


def _c_body(zb, y0b, y1b, y2b, disb, cb1, cb2, cb3,
            fw1t, fb1, fw2t, fb2, fw3t, fb3, g1t, gb1, g2t, gb2,
            gamma_o, beta_o, hacc):
    r = pl.program_id(0)

    @pl.when(r == 0)
    def _():
        hacc[...] = jnp.zeros((3, D), _f32)

    rows = lax.broadcasted_iota(_i32, (_RC, 1), 0) + r * _RC
    mask = rows < N
    for i, (yb, cb) in enumerate(((y0b, cb1), (y1b, cb2), (y2b, cb3))):
        t = (zb[i, 0] + zb[i, 1] + yb[...]) * disb[i][:, None] + cb[...]
        t = jnp.where(mask, jnp.maximum(t, 0.0), 0.0)
        hacc[pl.ds(i, 1), :] += jnp.sum(t, axis=0, keepdims=True)

    @pl.when(r == 15)
    def _():
        h0 = hacc[pl.ds(0, 1), :]
        h1 = hacc[pl.ds(1, 1), :]
        h2 = hacc[pl.ds(2, 1), :]
        a = (jnp.dot(h0, fw1t[pl.ds(0, D), :], preferred_element_type=_f32)
             + jnp.dot(h1, fw1t[pl.ds(D, D), :], preferred_element_type=_f32)
             + jnp.dot(h2, fw1t[pl.ds(2 * D, D), :],
                       preferred_element_type=_f32))
        a = jnp.maximum(a + fb1[...], 0.0)
        a = jnp.dot(a, fw2t[...], preferred_element_type=_f32) + fb2[...]
        a = jnp.maximum(a, 0.0)
        a = jnp.dot(a, fw3t[...], preferred_element_type=_f32) + fb3[...]
        m = jnp.max(a, axis=-1, keepdims=True)
        e = jnp.exp(a - m)
        p = e / jnp.sum(e, axis=-1, keepdims=True)
        x = (p[0:1, 0:1] * h0 + p[0:1, 1:2] * h1 + p[0:1, 2:3] * h2)
        gamma_o[...] = jnp.tanh(
            jnp.dot(x, g1t[...], preferred_element_type=_f32) + gb1[...])
        beta_o[...] = jnp.tanh(
            jnp.dot(x, g2t[...], preferred_element_type=_f32) + gb2[...])


def _c_call(z, ys, dis, cbs, att):
    full = lambda shape: pl.BlockSpec(shape, lambda r: tuple(0 for _ in shape))
    return pl.pallas_call(
        _c_body,
        grid=(16,),
        in_specs=[
            pl.BlockSpec((3, NC, _RC, D), lambda r: (0, 0, r, 0)),
            pl.BlockSpec((_RC, D), lambda r: (r, 0)),
            pl.BlockSpec((_RC, D), lambda r: (r, 0)),
            pl.BlockSpec((_RC, D), lambda r: (r, 0)),
            pl.BlockSpec((3, _RC), lambda r: (0, r)),
        ] + [full(a.shape) for a in cbs] + [full(a.shape) for a in att],
        out_specs=[full((1, D)), full((1, D))],
        out_shape=[jax.ShapeDtypeStruct((1, D), _f32),
                   jax.ShapeDtypeStruct((1, D), _f32)],
        scratch_shapes=[pltpu.VMEM((3, D), _f32)],
    )(z, ys[0], ys[1], ys[2], dis, *cbs, *att)


def kernel(x0, x1, x2, edge_index0, edge_index1, edge_index2,
           conv1_w, conv1_b, conv2_w, conv2_b, conv3_w, conv3_b,
           fc1_w, fc1_b, fc2_w, fc2_b,
           sa_fc1_w, sa_fc1_b, sa_fc2_w, sa_fc2_b, sa_fc3_w, sa_fc3_b):
    xp = [jnp.pad(x, ((0, NP - N), (0, 0))) for x in (x0, x1, x2)]

    pad = jnp.full((EP - E,), N, dtype=_i32)

    def edges(ei):
        return (jnp.concatenate([ei[0], pad]).reshape(NW, NCH, CH),
                jnp.concatenate([ei[1], pad]).reshape(NW, NCH, CH))

    s0, d0 = edges(edge_index0)
    s1, d1 = edges(edge_index1)
    s2, d2 = edges(edge_index2)
    src = jnp.stack([s0, s1, s2])
    dst = jnp.stack([d0, d1, d2])

    deg = _h_call(dst)
    deg_flat = deg.reshape(3, NC, NP)

    y0, y1, y2, dis = _a_call(xp, (conv1_w, conv2_w, conv3_w), deg_flat)

    z = _b_call(y0, y1, y2, src, dst)

    cbs = [conv1_b.reshape(1, D), conv2_b.reshape(1, D),
           conv3_b.reshape(1, D)]
    att = [sa_fc1_w.T, sa_fc1_b.reshape(1, D),
           sa_fc2_w.T, sa_fc2_b.reshape(1, D),
           sa_fc3_w.T, sa_fc3_b.reshape(1, 3),
           fc1_w.T, fc1_b.reshape(1, D),
           fc2_w.T, fc2_b.reshape(1, D)]
    gamma, beta = _c_call(z, (y0, y1, y2), dis, cbs, att)
    return gamma.reshape(D), beta.reshape(D)


# speedup vs baseline: 1.0519x; 1.0519x over previous
import functools

import jax
import jax.numpy as jnp
from jax import lax
from jax.experimental import pallas as pl
from jax.experimental.pallas import tpu as pltpu
from jax.experimental.pallas import tpu_sc as plsc

N = 10000
D = 128
E = 320000

NC = 2
NS = 16
NW = NC * NS

QH = 80
NP = QH * 128
CH = 128
NCH = 80
BI = 16
EPW = NCH * CH
TOTCH = NW * NCH
CF0 = 112
CF1 = 2 * NCH - CF0
EP = EPW * NW
RPT = NP // NS

_f32 = jnp.float32
_i32 = jnp.int32


def _h_body(dst_hbm, deg_hbm, hist_v, idxd_v, ident_v, hist_sh):
    c = lax.axis_index("c")
    s = lax.axis_index("s")
    w = c * NS + s
    zero16 = jnp.zeros((16,), _i32)
    one16 = jnp.ones((16,), _i32)

    @pl.loop(0, 5)
    def _(k):
        ident_v[0, pl.ds(k * 16, 16)] = (
            lax.broadcasted_iota(_i32, (16,), 0) + k * 16)

    for i in range(3):
        @pl.loop(0, QH)
        def _(q):
            @pl.loop(0, 8)
            def _(l):
                hist_v[q, pl.ds(l * 16, 16)] = zero16

        @pl.when(s == 0)
        def _():
            pltpu.sync_copy(hist_v, hist_sh)

        plsc.subcore_barrier()

        pltpu.sync_copy(dst_hbm.at[i, w], idxd_v)

        @pl.loop(0, NCH)
        def _(q):
            @pl.loop(0, 8)
            def _(l):
                v = idxd_v[q, pl.ds(l * 16, 16)]
                plsc.addupdate_scatter(hist_v, [v >> 7, v & 127], one16)

        pltpu.sync_copy(hist_v, hist_sh.at[ident_v.at[0]], add=True)
        plsc.subcore_barrier()

        @pl.when(s == 0)
        def _():
            pltpu.sync_copy(hist_sh, deg_hbm.at[i, c])

        plsc.subcore_barrier()


_h_call = functools.partial(
    pl.kernel,
    out_type=jax.ShapeDtypeStruct((3, NC, QH, 128), _i32),
    compiler_params=pltpu.CompilerParams(needs_layout_passes=False),
    mesh=plsc.VectorSubcoreMesh(core_axis_name="c", subcore_axis_name="s"),
    scratch_types=[
        pltpu.VMEM((QH, 128), _i32),
        pltpu.VMEM((NCH, 128), _i32),
        pltpu.VMEM((1, QH), _i32),
        pltpu.VMEM_SHARED((QH, 128), _i32),
    ],
)(_h_body)


def _a_body(x0b, x1b, x2b, w1, w2, w3, degb, y0b, y1b, y2b, disb):
    d = (degb[:, 0, :] + degb[:, 1, :] + 1).astype(_f32)
    dis = lax.rsqrt(d)
    disb[...] = dis
    for xb, wb, yb, i in ((x0b, w1, y0b, 0), (x1b, w2, y1b, 1),
                          (x2b, w3, y2b, 2)):
        xw = jnp.dot(xb[...], wb[...], preferred_element_type=_f32)
        yb[...] = xw * dis[i][:, None]


def _a_call(xp, ws, deg_flat):
    RA = NP // 8
    grid = 8
    return pl.pallas_call(
        _a_body,
        grid=(grid,),
        in_specs=[
            pl.BlockSpec((RA, D), lambda r: (r, 0)),
            pl.BlockSpec((RA, D), lambda r: (r, 0)),
            pl.BlockSpec((RA, D), lambda r: (r, 0)),
            pl.BlockSpec((D, D), lambda r: (0, 0)),
            pl.BlockSpec((D, D), lambda r: (0, 0)),
            pl.BlockSpec((D, D), lambda r: (0, 0)),
            pl.BlockSpec((3, NC, RA), lambda r: (0, 0, r)),
        ],
        out_specs=[
            pl.BlockSpec((RA, D), lambda r: (r, 0)),
            pl.BlockSpec((RA, D), lambda r: (r, 0)),
            pl.BlockSpec((RA, D), lambda r: (r, 0)),
            pl.BlockSpec((3, RA), lambda r: (0, r)),
        ],
        out_shape=[
            jax.ShapeDtypeStruct((NP, D), _f32),
            jax.ShapeDtypeStruct((NP, D), _f32),
            jax.ShapeDtypeStruct((NP, D), _f32),
            jax.ShapeDtypeStruct((3, NP), _f32),
        ],
    )(xp[0], xp[1], xp[2], ws[0], ws[1], ws[2], deg_flat)


def _b_body(y0, y1, y2, src_hbm, dst_hbm, out_hbm,
            idxs_v, idxd_v, buf_v, bufb_v, z_sh, sem, semb, semi):
    c = lax.axis_index("c")
    s = lax.axis_index("s")
    w = c * NS + s
    zero16 = jnp.zeros((16,), _f32)

    def zero_my_rows():
        @pl.loop(0, CH)
        def _(q):
            @pl.loop(0, 8)
            def _(l):
                buf_v[q, pl.ds(l * 16, 16)] = zero16

        @pl.loop(0, RPT // CH)
        def _(q):
            pltpu.sync_copy(buf_v, z_sh.at[pl.ds(s * RPT + q * CH, CH)])

    zero_my_rows()
    plsc.subcore_barrier()

    base_ch = jnp.where(c == 0, s * CF0, NS * CF0 + s * CF1)
    nblk_w = jnp.where(c == 0, CF0 // BI, CF1 // BI)

    for i, y_ref in enumerate((y0, y1, y2)):
        def stage(b, pb, sync):
            blk = pl.ds((base_ch + b * BI) * 1, BI)
            if sync:
                pltpu.sync_copy(src_hbm.at[i, blk], idxs_v.at[pb])
                pltpu.sync_copy(dst_hbm.at[i, blk], idxd_v.at[pb])
            else:
                pltpu.async_copy(src_hbm.at[i, blk], idxs_v.at[pb], semi)
                pltpu.async_copy(dst_hbm.at[i, blk], idxd_v.at[pb], semi)

        def wait_stage(pb):
            pltpu.make_async_copy(
                src_hbm.at[i, pl.ds(0, BI)], idxs_v.at[pb], semi).wait()
            pltpu.make_async_copy(
                dst_hbm.at[i, pl.ds(0, BI)], idxd_v.at[pb], semi).wait()

        def fire(pb, j, buf, sm):
            pltpu.async_copy(y_ref.at[idxs_v.at[pb, j]], buf, sm)

        def wait_g(buf, sm):
            pltpu.make_async_copy(
                y_ref.at[idxs_v.at[0, 0]], buf, sm).wait()

        def scat(pb, j, buf):
            pltpu.sync_copy(buf, z_sh.at[idxd_v.at[pb, j]], add=True)

        stage(0, 0, True)

        @pl.loop(0, nblk_w)
        def _(b):
            pb = b & 1

            @pl.when(b + 1 < nblk_w)
            def _():
                stage(b + 1, 1 - pb, False)

            fire(pb, 0, buf_v, sem)

            @pl.loop(0, BI // 2)
            def _(t):
                j = 2 * t
                fire(pb, j + 1, bufb_v, semb)
                wait_g(buf_v, sem)
                scat(pb, j, buf_v)

                @pl.when(j + 2 < BI)
                def _():
                    fire(pb, j + 2, buf_v, sem)

                wait_g(bufb_v, semb)
                scat(pb, j + 1, bufb_v)

            @pl.when(b + 1 < nblk_w)
            def _():
                wait_stage(1 - pb)

        plsc.subcore_barrier()

        for cc in range(NC):
            @pl.when(c == cc)
            def _():
                pltpu.sync_copy(z_sh.at[pl.ds(s * RPT, RPT)],
                                out_hbm.at[i, cc, pl.ds(s * RPT, RPT)])

        if i < 2:
            zero_my_rows()

        plsc.subcore_barrier()


_b_call = functools.partial(
    pl.kernel,
    out_type=jax.ShapeDtypeStruct((3, NC, NP, D), _f32),
    compiler_params=pltpu.CompilerParams(needs_layout_passes=False),
    mesh=plsc.VectorSubcoreMesh(core_axis_name="c", subcore_axis_name="s"),
    scratch_types=[
        pltpu.VMEM((2, BI, 128), _i32),
        pltpu.VMEM((2, BI, 128), _i32),
        pltpu.VMEM((CH, D), _f32),
        pltpu.VMEM((CH, D), _f32),
        pltpu.VMEM_SHARED((NP, D), _f32),
        pltpu.SemaphoreType.DMA,
        pltpu.SemaphoreType.DMA,
        pltpu.SemaphoreType.DMA,
    ],
)(_b_body)


_RC = NP // 16


def _c_body(zb, y0b, y1b, y2b, disb, cb1, cb2, cb3,
            fw1t, fb1, fw2t, fb2, fw3t, fb3, g1t, gb1, g2t, gb2,
            gamma_o, beta_o, hacc):
    r = pl.program_id(0)

    @pl.when(r == 0)
    def _():
        hacc[...] = jnp.zeros((3, D), _f32)

    rows = lax.broadcasted_iota(_i32, (_RC, 1), 0) + r * _RC
    mask = rows < N
    for i, (yb, cb) in enumerate(((y0b, cb1), (y1b, cb2), (y2b, cb3))):
        t = (zb[i, 0] + zb[i, 1] + yb[...]) * disb[i][:, None] + cb[...]
        t = jnp.where(mask, jnp.maximum(t, 0.0), 0.0)
        hacc[pl.ds(i, 1), :] += jnp.sum(t, axis=0, keepdims=True)

    @pl.when(r == 15)
    def _():
        h0 = hacc[pl.ds(0, 1), :]
        h1 = hacc[pl.ds(1, 1), :]
        h2 = hacc[pl.ds(2, 1), :]
        a = (jnp.dot(h0, fw1t[pl.ds(0, D), :], preferred_element_type=_f32)
             + jnp.dot(h1, fw1t[pl.ds(D, D), :], preferred_element_type=_f32)
             + jnp.dot(h2, fw1t[pl.ds(2 * D, D), :],
                       preferred_element_type=_f32))
        a = jnp.maximum(a + fb1[...], 0.0)
        a = jnp.dot(a, fw2t[...], preferred_element_type=_f32) + fb2[...]
        a = jnp.maximum(a, 0.0)
        a = jnp.dot(a, fw3t[...], preferred_element_type=_f32) + fb3[...]
        m = jnp.max(a, axis=-1, keepdims=True)
        e = jnp.exp(a - m)
        p = e / jnp.sum(e, axis=-1, keepdims=True)
        x = (p[0:1, 0:1] * h0 + p[0:1, 1:2] * h1 + p[0:1, 2:3] * h2)
        gamma_o[...] = jnp.tanh(
            jnp.dot(x, g1t[...], preferred_element_type=_f32) + gb1[...])
        beta_o[...] = jnp.tanh(
            jnp.dot(x, g2t[...], preferred_element_type=_f32) + gb2[...])


def _c_call(z, ys, dis, cbs, att):
    full = lambda shape: pl.BlockSpec(shape, lambda r: tuple(0 for _ in shape))
    return pl.pallas_call(
        _c_body,
        grid=(16,),
        in_specs=[
            pl.BlockSpec((3, NC, _RC, D), lambda r: (0, 0, r, 0)),
            pl.BlockSpec((_RC, D), lambda r: (r, 0)),
            pl.BlockSpec((_RC, D), lambda r: (r, 0)),
            pl.BlockSpec((_RC, D), lambda r: (r, 0)),
            pl.BlockSpec((3, _RC), lambda r: (0, r)),
        ] + [full(a.shape) for a in cbs] + [full(a.shape) for a in att],
        out_specs=[full((1, D)), full((1, D))],
        out_shape=[jax.ShapeDtypeStruct((1, D), _f32),
                   jax.ShapeDtypeStruct((1, D), _f32)],
        scratch_shapes=[pltpu.VMEM((3, D), _f32)],
    )(z, ys[0], ys[1], ys[2], dis, *cbs, *att)


def kernel(x0, x1, x2, edge_index0, edge_index1, edge_index2,
           conv1_w, conv1_b, conv2_w, conv2_b, conv3_w, conv3_b,
           fc1_w, fc1_b, fc2_w, fc2_b,
           sa_fc1_w, sa_fc1_b, sa_fc2_w, sa_fc2_b, sa_fc3_w, sa_fc3_b):
    xp = [jnp.pad(x, ((0, NP - N), (0, 0))) for x in (x0, x1, x2)]

    pad = jnp.full((EP - E,), N, dtype=_i32)

    def edges(ei):
        return (jnp.concatenate([ei[0], pad]).reshape(NW, NCH, CH),
                jnp.concatenate([ei[1], pad]).reshape(NW, NCH, CH))

    s0, d0 = edges(edge_index0)
    s1, d1 = edges(edge_index1)
    s2, d2 = edges(edge_index2)
    src = jnp.stack([s0, s1, s2])
    dst = jnp.stack([d0, d1, d2])

    deg = _h_call(dst)
    deg_flat = deg.reshape(3, NC, NP)

    y0, y1, y2, dis = _a_call(xp, (conv1_w, conv2_w, conv3_w), deg_flat)

    z = _b_call(y0, y1, y2, src.reshape(3, TOTCH, CH),
                dst.reshape(3, TOTCH, CH))

    cbs = [conv1_b.reshape(1, D), conv2_b.reshape(1, D),
           conv3_b.reshape(1, D)]
    att = [sa_fc1_w.T, sa_fc1_b.reshape(1, D),
           sa_fc2_w.T, sa_fc2_b.reshape(1, D),
           sa_fc3_w.T, sa_fc3_b.reshape(1, 3),
           fc1_w.T, fc1_b.reshape(1, D),
           fc2_w.T, fc2_b.reshape(1, D)]
    gamma, beta = _c_call(z, (y0, y1, y2), dis, cbs, att)
    return gamma.reshape(D), beta.reshape(D)


# speedup vs baseline: 1.3899x; 1.3213x over previous
import functools

import jax
import jax.numpy as jnp
from jax import lax
from jax.experimental import pallas as pl
from jax.experimental.pallas import tpu as pltpu
from jax.experimental.pallas import tpu_sc as plsc

N = 10000
D = 128
E = 320000

NC = 2
NS = 16
NW = NC * NS

QH = 80
NP = QH * 128
CH = 128
NCH = 80
BI = 16
EPW = NCH * CH
TOTCH = NW * NCH
NCHW = TOTCH // NS
NBLKW = NCHW // BI
EP = EPW * NW
RPT = NP // NS
COLS = D // 2

_f32 = jnp.float32
_i32 = jnp.int32


def _h_body(dst_hbm, deg_hbm, hist_v, idxd_v, ident_v, hist_sh):
    c = lax.axis_index("c")
    s = lax.axis_index("s")
    w = c * NS + s
    zero16 = jnp.zeros((16,), _i32)
    one16 = jnp.ones((16,), _i32)

    @pl.loop(0, 5)
    def _(k):
        ident_v[0, pl.ds(k * 16, 16)] = (
            lax.broadcasted_iota(_i32, (16,), 0) + k * 16)

    for i in range(3):
        @pl.loop(0, QH)
        def _(q):
            @pl.loop(0, 8)
            def _(l):
                hist_v[q, pl.ds(l * 16, 16)] = zero16

        @pl.when(s == 0)
        def _():
            pltpu.sync_copy(hist_v, hist_sh)

        plsc.subcore_barrier()

        pltpu.sync_copy(dst_hbm.at[i, w], idxd_v)

        @pl.loop(0, NCH)
        def _(q):
            @pl.loop(0, 8)
            def _(l):
                v = idxd_v[q, pl.ds(l * 16, 16)]
                plsc.addupdate_scatter(hist_v, [v >> 7, v & 127], one16)

        pltpu.sync_copy(hist_v, hist_sh.at[ident_v.at[0]], add=True)
        plsc.subcore_barrier()

        @pl.when(s == 0)
        def _():
            pltpu.sync_copy(hist_sh, deg_hbm.at[i, c])

        plsc.subcore_barrier()


_h_call = functools.partial(
    pl.kernel,
    out_type=jax.ShapeDtypeStruct((3, NC, QH, 128), _i32),
    compiler_params=pltpu.CompilerParams(needs_layout_passes=False),
    mesh=plsc.VectorSubcoreMesh(core_axis_name="c", subcore_axis_name="s"),
    scratch_types=[
        pltpu.VMEM((QH, 128), _i32),
        pltpu.VMEM((NCH, 128), _i32),
        pltpu.VMEM((1, QH), _i32),
        pltpu.VMEM_SHARED((QH, 128), _i32),
    ],
)(_h_body)


def _a_body(x0b, x1b, x2b, w1, w2, w3, degb, yab, ybb, disb):
    d = (degb[:, 0, :] + degb[:, 1, :] + 1).astype(_f32)
    dis = lax.rsqrt(d)
    disb[...] = dis
    for xb, wb, i in ((x0b, w1, 0), (x1b, w2, 1), (x2b, w3, 2)):
        xw = jnp.dot(xb[...], wb[...], preferred_element_type=_f32)
        y = xw * dis[i][:, None]
        yab[i] = y[:, :COLS]
        ybb[i] = y[:, COLS:]


def _a_call(xp, ws, deg_flat):
    RA = NP // 8
    grid = 8
    return pl.pallas_call(
        _a_body,
        grid=(grid,),
        in_specs=[
            pl.BlockSpec((RA, D), lambda r: (r, 0)),
            pl.BlockSpec((RA, D), lambda r: (r, 0)),
            pl.BlockSpec((RA, D), lambda r: (r, 0)),
            pl.BlockSpec((D, D), lambda r: (0, 0)),
            pl.BlockSpec((D, D), lambda r: (0, 0)),
            pl.BlockSpec((D, D), lambda r: (0, 0)),
            pl.BlockSpec((3, NC, RA), lambda r: (0, 0, r)),
        ],
        out_specs=[
            pl.BlockSpec((3, RA, COLS), lambda r: (0, r, 0)),
            pl.BlockSpec((3, RA, COLS), lambda r: (0, r, 0)),
            pl.BlockSpec((3, RA), lambda r: (0, r)),
        ],
        out_shape=[
            jax.ShapeDtypeStruct((3, NP, COLS), _f32),
            jax.ShapeDtypeStruct((3, NP, COLS), _f32),
            jax.ShapeDtypeStruct((3, NP), _f32),
        ],
    )(xp[0], xp[1], xp[2], ws[0], ws[1], ws[2], deg_flat)


def _b_body(ya, yb, src_hbm, dst_hbm, out_hbm,
            idxs_v, idxd_v, buf_v, bufb_v, y_sp, z_sh, sem, semb, semi):
    c = lax.axis_index("c")
    s = lax.axis_index("s")
    zero16 = jnp.zeros((16,), _f32)
    base_ch = s * NCHW

    def zero_my_rows():
        @pl.loop(0, CH)
        def _(q):
            @pl.loop(0, COLS // 16)
            def _(l):
                buf_v[q, pl.ds(l * 16, 16)] = zero16

        @pl.loop(0, RPT // CH)
        def _(q):
            pltpu.sync_copy(buf_v, z_sh.at[pl.ds(s * RPT + q * CH, CH)])

    zero_my_rows()
    plsc.subcore_barrier()

    for i in range(3):
        def stage(b, pb, sync):
            blk = pl.ds(base_ch + b * BI, BI)
            if sync:
                pltpu.sync_copy(src_hbm.at[i, blk], idxs_v.at[pb])
                pltpu.sync_copy(dst_hbm.at[i, blk], idxd_v.at[pb])
            else:
                pltpu.async_copy(src_hbm.at[i, blk], idxs_v.at[pb], semi)
                pltpu.async_copy(dst_hbm.at[i, blk], idxd_v.at[pb], semi)

        def wait_stage(pb):
            pltpu.make_async_copy(
                src_hbm.at[i, pl.ds(0, BI)], idxs_v.at[pb], semi).wait()
            pltpu.make_async_copy(
                dst_hbm.at[i, pl.ds(0, BI)], idxd_v.at[pb], semi).wait()

        def run_edges(table):
            def fire(pb, j, buf, sm):
                pltpu.async_copy(table.at[idxs_v.at[pb, j]], buf, sm)

            def wait_g(buf, sm):
                pltpu.make_async_copy(
                    table.at[idxs_v.at[0, 0]], buf, sm).wait()

            def scat(pb, j, buf):
                pltpu.sync_copy(buf, z_sh.at[idxd_v.at[pb, j]], add=True)

            stage(0, 0, True)

            @pl.loop(0, NBLKW)
            def _(b):
                pb = b & 1

                @pl.when(b + 1 < NBLKW)
                def _():
                    stage(b + 1, 1 - pb, False)

                fire(pb, 0, buf_v, sem)

                @pl.loop(0, BI // 2)
                def _(t):
                    j = 2 * t
                    fire(pb, j + 1, bufb_v, semb)
                    wait_g(buf_v, sem)
                    scat(pb, j, buf_v)

                    @pl.when(j + 2 < BI)
                    def _():
                        fire(pb, j + 2, buf_v, sem)

                    wait_g(bufb_v, semb)
                    scat(pb, j + 1, bufb_v)

                @pl.when(b + 1 < NBLKW)
                def _():
                    wait_stage(1 - pb)

        @pl.when(c == 1)
        def _():
            pltpu.sync_copy(yb.at[i, pl.ds(s * RPT, RPT)],
                            y_sp.at[pl.ds(s * RPT, RPT)])

        plsc.subcore_barrier()

        @pl.when(c == 0)
        def _():
            run_edges(ya.at[i])

        @pl.when(c == 1)
        def _():
            run_edges(y_sp)

        plsc.subcore_barrier()

        for cc in range(NC):
            @pl.when(c == cc)
            def _():
                pltpu.sync_copy(z_sh.at[pl.ds(s * RPT, RPT)],
                                out_hbm.at[i, cc, pl.ds(s * RPT, RPT)])

        if i < 2:
            zero_my_rows()

        plsc.subcore_barrier()


_b_call = functools.partial(
    pl.kernel,
    out_type=jax.ShapeDtypeStruct((3, NC, NP, COLS), _f32),
    compiler_params=pltpu.CompilerParams(needs_layout_passes=False,
                                         use_tc_tiling_on_sc=False),
    mesh=plsc.VectorSubcoreMesh(core_axis_name="c", subcore_axis_name="s"),
    scratch_types=[
        pltpu.VMEM((2, BI, 128), _i32),
        pltpu.VMEM((2, BI, 128), _i32),
        pltpu.VMEM((CH, COLS), _f32),
        pltpu.VMEM((CH, COLS), _f32),
        pltpu.VMEM_SHARED((NP, COLS), _f32),
        pltpu.VMEM_SHARED((NP, COLS), _f32),
        pltpu.SemaphoreType.DMA,
        pltpu.SemaphoreType.DMA,
        pltpu.SemaphoreType.DMA,
    ],
)(_b_body)


_RC = NP // 16


def _c_body(zb, yab, ybb, disb, cb1, cb2, cb3,
            fw1t, fb1, fw2t, fb2, fw3t, fb3, g1t, gb1, g2t, gb2,
            gamma_o, beta_o, hacc0, hacc1):
    r = pl.program_id(0)

    @pl.when(r == 0)
    def _():
        hacc0[...] = jnp.zeros((3, COLS), _f32)
        hacc1[...] = jnp.zeros((3, COLS), _f32)

    rows = lax.broadcasted_iota(_i32, (_RC, 1), 0) + r * _RC
    mask = rows < N
    for i, cb in enumerate((cb1, cb2, cb3)):
        d = disb[i][:, None]
        t0 = (zb[i, 0] + yab[i]) * d + cb[:, :COLS]
        t1 = (zb[i, 1] + ybb[i]) * d + cb[:, COLS:]
        t0 = jnp.where(mask, jnp.maximum(t0, 0.0), 0.0)
        t1 = jnp.where(mask, jnp.maximum(t1, 0.0), 0.0)
        hacc0[pl.ds(i, 1), :] += jnp.sum(t0, axis=0, keepdims=True)
        hacc1[pl.ds(i, 1), :] += jnp.sum(t1, axis=0, keepdims=True)

    @pl.when(r == 15)
    def _():
        def h(i):
            return jnp.concatenate(
                [hacc0[pl.ds(i, 1), :], hacc1[pl.ds(i, 1), :]], axis=1)

        h0, h1, h2 = h(0), h(1), h(2)
        a = (jnp.dot(h0, fw1t[pl.ds(0, D), :], preferred_element_type=_f32)
             + jnp.dot(h1, fw1t[pl.ds(D, D), :], preferred_element_type=_f32)
             + jnp.dot(h2, fw1t[pl.ds(2 * D, D), :],
                       preferred_element_type=_f32))
        a = jnp.maximum(a + fb1[...], 0.0)
        a = jnp.dot(a, fw2t[...], preferred_element_type=_f32) + fb2[...]
        a = jnp.maximum(a, 0.0)
        a = jnp.dot(a, fw3t[...], preferred_element_type=_f32) + fb3[...]
        m = jnp.max(a, axis=-1, keepdims=True)
        e = jnp.exp(a - m)
        p = e / jnp.sum(e, axis=-1, keepdims=True)
        x = (p[0:1, 0:1] * h0 + p[0:1, 1:2] * h1 + p[0:1, 2:3] * h2)
        gamma_o[...] = jnp.tanh(
            jnp.dot(x, g1t[...], preferred_element_type=_f32) + gb1[...])
        beta_o[...] = jnp.tanh(
            jnp.dot(x, g2t[...], preferred_element_type=_f32) + gb2[...])


def _c_call(z, ya, yb, dis, cbs, att):
    full = lambda shape: pl.BlockSpec(shape, lambda r: tuple(0 for _ in shape))
    return pl.pallas_call(
        _c_body,
        grid=(16,),
        in_specs=[
            pl.BlockSpec((3, NC, _RC, COLS), lambda r: (0, 0, r, 0)),
            pl.BlockSpec((3, _RC, COLS), lambda r: (0, r, 0)),
            pl.BlockSpec((3, _RC, COLS), lambda r: (0, r, 0)),
            pl.BlockSpec((3, _RC), lambda r: (0, r)),
        ] + [full(a.shape) for a in cbs] + [full(a.shape) for a in att],
        out_specs=[full((1, D)), full((1, D))],
        out_shape=[jax.ShapeDtypeStruct((1, D), _f32),
                   jax.ShapeDtypeStruct((1, D), _f32)],
        scratch_shapes=[pltpu.VMEM((3, COLS), _f32),
                        pltpu.VMEM((3, COLS), _f32)],
    )(z, ya, yb, dis, *cbs, *att)


def kernel(x0, x1, x2, edge_index0, edge_index1, edge_index2,
           conv1_w, conv1_b, conv2_w, conv2_b, conv3_w, conv3_b,
           fc1_w, fc1_b, fc2_w, fc2_b,
           sa_fc1_w, sa_fc1_b, sa_fc2_w, sa_fc2_b, sa_fc3_w, sa_fc3_b):
    xp = [jnp.pad(x, ((0, NP - N), (0, 0))) for x in (x0, x1, x2)]

    pad = jnp.full((EP - E,), N, dtype=_i32)

    def edges(ei):
        return (jnp.concatenate([ei[0], pad]).reshape(NW, NCH, CH),
                jnp.concatenate([ei[1], pad]).reshape(NW, NCH, CH))

    s0, d0 = edges(edge_index0)
    s1, d1 = edges(edge_index1)
    s2, d2 = edges(edge_index2)
    src = jnp.stack([s0, s1, s2])
    dst = jnp.stack([d0, d1, d2])

    deg = _h_call(dst)
    deg_flat = deg.reshape(3, NC, NP)

    ya, yb, dis = _a_call(xp, (conv1_w, conv2_w, conv3_w), deg_flat)

    z = _b_call(ya, yb, src.reshape(3, TOTCH, CH),
                dst.reshape(3, TOTCH, CH))

    cbs = [conv1_b.reshape(1, D), conv2_b.reshape(1, D),
           conv3_b.reshape(1, D)]
    att = [sa_fc1_w.T, sa_fc1_b.reshape(1, D),
           sa_fc2_w.T, sa_fc2_b.reshape(1, D),
           sa_fc3_w.T, sa_fc3_b.reshape(1, 3),
           fc1_w.T, fc1_b.reshape(1, D),
           fc2_w.T, fc2_b.reshape(1, D)]
    gamma, beta = _c_call(z, ya, yb, dis, cbs, att)
    return gamma.reshape(D), beta.reshape(D)
